# baseline (speedup 1.0000x reference)
import functools

import jax
import jax.numpy as jnp
from jax.experimental import pallas as pl


def _node_kernel(h_ref, xyz_ref, wo1_ref, bo1_ref, wo2_ref, bo2_ref,
                 we1c_ref, we1h_ref, be1_ref, g_ref, b_ref):
    h = h_ref[...]
    xyz = xyz_ref[...]
    t = jnp.maximum(
        jnp.dot(h, wo1_ref[...], preferred_element_type=jnp.float32)
        + bo1_ref[...], 0.0)
    off = jnp.dot(t, wo2_ref[...], preferred_element_type=jnp.float32) + bo2_ref[...]
    c = xyz + off
    we1c = we1c_ref[...]
    g_ref[...] = (jnp.dot(h, we1h_ref[...], preferred_element_type=jnp.float32)
                  + jnp.dot(xyz, we1c, preferred_element_type=jnp.float32)
                  + be1_ref[...])
    b_ref[...] = jnp.dot(c, we1c, preferred_element_type=jnp.float32)


def _edge_kernel(e1_ref, we2_ref, be2_ref, e2_ref):
    e2_ref[...] = jnp.maximum(
        jnp.dot(e1_ref[...], we2_ref[...], preferred_element_type=jnp.float32)
        + be2_ref[...], 0.0)


def _update_kernel(h_ref, agg_ref, wu1_ref, bu1_ref, wu2_ref, bu2_ref, out_ref):
    t = jnp.maximum(
        jnp.dot(agg_ref[...], wu1_ref[...], preferred_element_type=jnp.float32)
        + bu1_ref[...], 0.0)
    out_ref[...] = h_ref[...] + (
        jnp.dot(t, wu2_ref[...], preferred_element_type=jnp.float32)
        + bu2_ref[...])


def kernel(x, xyz, edge_index, Wo1, bo1, Wo2, bo2, We1, be1, We2, be2,
           Wu1, bu1, Wu2, bu2):
    src = edge_index[0]
    dst = edge_index[1]
    n, d = x.shape
    e_num = src.shape[0]
    hdim = We2.shape[-1]
    num_layers = Wo1.shape[0]

    BN = 400
    BE = 2000

    h = x
    full = lambda shape: pl.BlockSpec(shape, lambda i: (0,) * len(shape))

    node_call = pl.pallas_call(
        _node_kernel,
        grid=(n // BN,),
        in_specs=[
            pl.BlockSpec((BN, d), lambda i: (i, 0)),
            pl.BlockSpec((BN, 3), lambda i: (i, 0)),
            full(Wo1.shape[1:]), full((1, bo1.shape[-1])),
            full(Wo2.shape[1:]), full((1, 3)),
            full((3, hdim)), full((d, hdim)), full((1, hdim)),
        ],
        out_specs=[
            pl.BlockSpec((BN, hdim), lambda i: (i, 0)),
            pl.BlockSpec((BN, hdim), lambda i: (i, 0)),
        ],
        out_shape=[
            jax.ShapeDtypeStruct((n, hdim), jnp.float32),
            jax.ShapeDtypeStruct((n, hdim), jnp.float32),
        ],
    )

    edge_call = pl.pallas_call(
        _edge_kernel,
        grid=(e_num // BE,),
        in_specs=[
            pl.BlockSpec((BE, hdim), lambda i: (i, 0)),
            full((hdim, hdim)), full((1, hdim)),
        ],
        out_specs=pl.BlockSpec((BE, hdim), lambda i: (i, 0)),
        out_shape=jax.ShapeDtypeStruct((e_num, hdim), jnp.float32),
    )

    update_call = pl.pallas_call(
        _update_kernel,
        grid=(n // BN,),
        in_specs=[
            pl.BlockSpec((BN, d), lambda i: (i, 0)),
            pl.BlockSpec((BN, hdim), lambda i: (i, 0)),
            full((hdim, hdim)), full((1, hdim)),
            full((hdim, d)), full((1, d)),
        ],
        out_specs=pl.BlockSpec((BN, d), lambda i: (i, 0)),
        out_shape=jax.ShapeDtypeStruct((n, d), jnp.float32),
    )

    for l in range(num_layers):
        g, b = node_call(
            h, xyz, Wo1[l], bo1[l][None], Wo2[l], bo2[l][None],
            We1[l, :3], We1[l, 3:], be1[l][None])
        e1 = jnp.maximum(g[src] - b[dst], 0.0)
        e2 = edge_call(e1, We2[l], be2[l][None])
        agg = jnp.zeros((n, hdim), jnp.float32).at[dst].max(e2)
        h = update_call(h, agg, Wu1[l], bu1[l][None], Wu2[l], bu2[l][None])
    return h
